# baseline (speedup 1.0000x reference)
import functools

import jax
import jax.numpy as jnp
from jax import lax
from jax.experimental import pallas as pl
from jax.experimental.pallas import tpu as pltpu
from jax.experimental.pallas import tpu_sc as plsc

N = 10000
E = 320000
NW = 32
BS = 320
NPAD = NW * BS
SENT = BS
SLICE = E // NW
SCAP = 14080
CAPW = 325632
KG = 128
KB = 2048

_mesh = plsc.VectorSubcoreMesh(core_axis_name="c", subcore_axis_name="s")


def _wid():
    return lax.axis_index("s") * 2 + lax.axis_index("c")


@functools.partial(
    pl.kernel,
    mesh=_mesh,
    out_type=[
        jax.ShapeDtypeStruct((NW * SCAP,), jnp.int32),
        jax.ShapeDtypeStruct((NW * SCAP,), jnp.int32),
        jax.ShapeDtypeStruct((NW, 64), jnp.int32),
    ],
    scratch_types=[
        pltpu.VMEM((SLICE,), jnp.int32),
        pltpu.VMEM((SLICE,), jnp.int32),
        pltpu.VMEM((SCAP,), jnp.int32),
        pltpu.VMEM((SCAP,), jnp.int32),
        pltpu.VMEM((32,), jnp.int32),
        pltpu.VMEM((32,), jnp.int32),
        pltpu.VMEM((32,), jnp.int32),
        pltpu.VMEM((64,), jnp.int32),
    ],
    compiler_params=pltpu.CompilerParams(needs_layout_passes=False),
)
def _p0a(src_hbm, dst_hbm, ssrc_out, sdl_out, tabs_out,
         sbuf, dbuf, stsrc, stdl, hist, starts, offs, tab):
    wid = _wid()
    lane = lax.broadcasted_iota(jnp.int32, (16,), 0)
    zero16 = jnp.zeros((16,), jnp.int32)
    sent16 = jnp.full((16,), SENT, jnp.int32)
    ones16 = jnp.ones((16,), jnp.int32)

    pltpu.sync_copy(src_hbm.at[pl.ds(wid * SLICE, SLICE)], sbuf)
    pltpu.sync_copy(dst_hbm.at[pl.ds(wid * SLICE, SLICE)], dbuf)

    hist[pl.ds(0, 16)] = zero16
    hist[pl.ds(16, 16)] = zero16
    offs[pl.ds(0, 16)] = zero16
    offs[pl.ds(16, 16)] = zero16

    def _sent(i, _):
        stsrc[pl.ds(i * 16, 16)] = zero16
        stdl[pl.ds(i * 16, 16)] = sent16
        return 0

    lax.fori_loop(0, SCAP // 16, _sent, 0)

    def _count(g, _):
        bv = dbuf[pl.ds(g * 16, 16)] // BS
        plsc.addupdate_scatter(hist, [bv], ones16)
        return 0

    lax.fori_loop(0, SLICE // 16, _count, 0)

    h0 = hist[pl.ds(0, 16)]
    h1 = hist[pl.ds(16, 16)]
    l0 = ((h0 + 127) // 128) * 128
    l1 = ((h1 + 127) // 128) * 128
    c0 = jnp.cumsum(l0)
    c1 = jnp.cumsum(l1) + c0[15]
    starts[pl.ds(0, 16)] = c0 - l0
    starts[pl.ds(16, 16)] = c1 - l1
    tab[pl.ds(0, 16)] = l0
    tab[pl.ds(16, 16)] = l1
    tab[pl.ds(32, 16)] = c0 - l0
    tab[pl.ds(48, 16)] = c1 - l1
    pltpu.sync_copy(tab, tabs_out.at[wid])

    def _scatter(g, _):
        dv = dbuf[pl.ds(g * 16, 16)]
        sv = sbuf[pl.ds(g * 16, 16)]
        bv = dv // BS
        dlv = dv - bv * BS
        st = plsc.load_gather(starts, [bv])
        of = plsc.load_gather(offs, [bv])
        rk, _lm = plsc.scan_count(bv)
        pos = st + of + rk - 1
        plsc.store_scatter(stsrc, [pos], sv)
        plsc.store_scatter(stdl, [pos], dlv)
        plsc.addupdate_scatter(offs, [bv], ones16)
        return 0

    lax.fori_loop(0, SLICE // 16, _scatter, 0)

    pltpu.sync_copy(stsrc, ssrc_out.at[pl.ds(wid * SCAP, SCAP)])
    pltpu.sync_copy(stdl, sdl_out.at[pl.ds(wid * SCAP, SCAP)])


@functools.partial(
    pl.kernel,
    mesh=_mesh,
    out_type=[
        jax.ShapeDtypeStruct((NW * CAPW,), jnp.int32),
        jax.ShapeDtypeStruct((NW * CAPW,), jnp.int32),
        jax.ShapeDtypeStruct((NW, 16), jnp.int32),
    ],
    scratch_types=[
        pltpu.VMEM((NW, 64), jnp.int32),
        pltpu.VMEM((128,), jnp.int32),
        pltpu.VMEM((128,), jnp.int32),
        pltpu.VMEM((16,), jnp.int32),
        pltpu.SemaphoreType.DMA,
    ],
    compiler_params=pltpu.CompilerParams(needs_layout_passes=False),
)
def _p0b(ssrc, sdl, tabs, srcs_out, dls_out, cnts_out,
         tabv, bsrc, bdl, cvec, sem):
    wid = _wid()
    lane = lax.broadcasted_iota(jnp.int32, (16,), 0)
    zero16 = jnp.zeros((16,), jnp.int32)
    sent16 = jnp.full((16,), SENT, jnp.int32)
    pltpu.sync_copy(tabs, tabv)
    for i in range(8):
        bsrc[pl.ds(i * 16, 16)] = zero16
        bdl[pl.ds(i * 16, 16)] = sent16

    wsp = jnp.full((16,), 0, jnp.int32) + wid
    l_a = plsc.load_gather(tabv, [lane, wsp])
    l_b = plsc.load_gather(tabv, [lane + 16, wsp])
    s_a = plsc.load_gather(tabv, [lane, wsp + 32])
    s_b = plsc.load_gather(tabv, [lane + 16, wsp + 32])

    off = 0
    wbase = pl.multiple_of(wid * CAPW, 128)
    for s in range(NW):
        ln = (l_a if s < 16 else l_b)[s % 16]
        st = pl.multiple_of((s_a if s < 16 else s_b)[s % 16] + s * SCAP, 128)
        n128 = ln // 128
        off = pl.multiple_of(off, 128)

        def _fire(k, _):
            dsto = pl.multiple_of(wbase + off + k * 128, 128)
            srco = pl.multiple_of(st + k * 128, 128)
            pltpu.async_copy(ssrc.at[pl.ds(srco, 128)],
                             srcs_out.at[pl.ds(dsto, 128)], sem)
            pltpu.async_copy(sdl.at[pl.ds(srco, 128)],
                             dls_out.at[pl.ds(dsto, 128)], sem)
            return 0

        def _drain(k, _):
            pltpu.make_async_copy(ssrc.at[pl.ds(0, 128)],
                                  srcs_out.at[pl.ds(0, 128)], sem).wait()
            pltpu.make_async_copy(ssrc.at[pl.ds(0, 128)],
                                  srcs_out.at[pl.ds(0, 128)], sem).wait()
            return 0

        lax.fori_loop(0, n128, _fire, 0)
        lax.fori_loop(0, n128, _drain, 0)
        off = off + ln

    tail = pl.multiple_of(wbase + off, 128)
    pltpu.sync_copy(bsrc, srcs_out.at[pl.ds(tail, 128)])
    pltpu.sync_copy(bdl, dls_out.at[pl.ds(tail, 128)])
    cvec[...] = jnp.where(lane == 0, off, 0)
    pltpu.sync_copy(cvec, cnts_out.at[wid])


def _bucket_edges(src, dst):
    ssrc, sdl, tabs = _p0a(src, dst)
    srcs, dls, cnts = _p0b(ssrc, sdl, tabs)
    return srcs.reshape(NW, CAPW), dls.reshape(NW, CAPW), cnts


def _make_segmax(F, KBF, NC):
    @functools.partial(
        pl.kernel,
        mesh=_mesh,
        out_type=jax.ShapeDtypeStruct((NPAD, F), jnp.float32),
        scratch_types=(
            [pltpu.VMEM((KBF,), jnp.int32),
             pltpu.VMEM((KBF,), jnp.int32),
             pltpu.VMEM((KBF, F), jnp.float32)]
            + [pltpu.VMEM((BS + 16, F), jnp.float32) for _ in range(NC)]
            + [pltpu.VMEM((16,), jnp.int32),
               pltpu.SemaphoreType.DMA]
        ),
        compiler_params=pltpu.CompilerParams(needs_layout_passes=False,
                                             use_tc_tiling_on_sc=False),
    )
    def segmax(h_hbm, srcs, dls, cnts, out_hbm, cidx, cdl, rows, *rest):
        acc = rest[:NC]
        cvec, sem = rest[NC], rest[NC + 1]
        wid = _wid()
        lo = wid * BS
        neg16 = jnp.full((16,), -jnp.inf, jnp.float32)

        def _initrow(i, _):
            for c in range(NC):
                for f in range(F // 16):
                    acc[c][i, pl.ds(f * 16, 16)] = neg16
            return 0

        lax.fori_loop(0, BS + 16, _initrow, 0)

        pltpu.sync_copy(cnts.at[wid], cvec)
        total = jnp.max(cvec[...])
        nbig = (total + KBF - 1) // KBF

        def _big(cb, _):
            pltpu.sync_copy(srcs.at[wid, pl.ds(cb * KBF, KBF)], cidx)
            pltpu.sync_copy(dls.at[wid, pl.ds(cb * KBF, KBF)], cdl)
            nsm = jnp.minimum(KBF // KG, (total - cb * KBF + KG - 1) // KG)

            def _fire(k, _):
                pltpu.async_copy(h_hbm.at[cidx.at[pl.ds(k * KG, KG)]],
                                 rows.at[pl.ds(k * KG, KG)], sem)
                return 0

            def _drain(k, _):
                pltpu.make_async_copy(h_hbm.at[pl.ds(0, KG)],
                                      rows.at[pl.ds(k * KG, KG)], sem).wait()
                return 0

            lax.fori_loop(0, nsm, _fire, 0)
            lax.fori_loop(0, nsm, _drain, 0)

            def _edge16(g, _):
                gb = g * 16
                dlv = cdl[pl.ds(gb, 16)]
                for j in range(16):
                    a = acc[j % NC]
                    dl = dlv[j]
                    for f in range(F // 16):
                        cur = a[dl, pl.ds(f * 16, 16)]
                        r = rows[gb + j, pl.ds(f * 16, 16)]
                        a[dl, pl.ds(f * 16, 16)] = jnp.maximum(cur, r)
                return 0

            lax.fori_loop(0, nsm * (KG // 16), _edge16, 0)
            return 0

        lax.fori_loop(0, nbig, _big, 0)

        def _fixrow(i, _):
            for f in range(F // 16):
                v = acc[0][i, pl.ds(f * 16, 16)]
                for c in range(1, NC):
                    v = jnp.maximum(v, acc[c][i, pl.ds(f * 16, 16)])
                acc[0][i, pl.ds(f * 16, 16)] = jnp.where(v == -jnp.inf,
                                                         0.0, v)
            return 0

        lax.fori_loop(0, BS, _fixrow, 0)
        pltpu.sync_copy(acc[0].at[pl.ds(0, BS)], out_hbm.at[pl.ds(lo, BS)])

    return segmax


_segmax128 = _make_segmax(128, 256, 2)
_segmax16 = _make_segmax(16, 2048, 8)


def _linear(agg, h, Wl, Wr, b, act):
    M, F = h.shape
    H = Wl.shape[1]
    BM = 1000

    def body(agg_ref, h_ref, wl_ref, wr_ref, b_ref, o_ref):
        o = jnp.dot(agg_ref[...], wl_ref[...],
                    preferred_element_type=jnp.float32)
        o = o + jnp.dot(h_ref[...], wr_ref[...],
                        preferred_element_type=jnp.float32)
        o = o + b_ref[...]
        if act == "relu":
            o = jnp.maximum(o, 0.0)
        elif act == "lsm":
            mx = jnp.max(o, axis=1, keepdims=True)
            e = jnp.exp(o - mx)
            s = jnp.sum(e, axis=1, keepdims=True)
            o = o - mx - jnp.log(s)
        o_ref[...] = o

    return pl.pallas_call(
        body,
        grid=(M // BM,),
        in_specs=[
            pl.BlockSpec((BM, F), lambda i: (i, 0)),
            pl.BlockSpec((BM, F), lambda i: (i, 0)),
            pl.BlockSpec((F, H), lambda i: (0, 0)),
            pl.BlockSpec((F, H), lambda i: (0, 0)),
            pl.BlockSpec((1, H), lambda i: (0, 0)),
        ],
        out_specs=pl.BlockSpec((BM, H), lambda i: (i, 0)),
        out_shape=jax.ShapeDtypeStruct((M, H), jnp.float32),
    )(agg, h, Wl, Wr, b[None])


def kernel(x, edge_index, Wl1, Wr1, b1, Wl2, Wr2, b2, Wl3, Wr3, b3,
           Wl4, Wr4, b4, Wl5, Wr5, b5, Wl6, Wr6, b6, Wl7, Wr7, b7):
    src = edge_index[0]
    dst = edge_index[1]
    srcs, dls, cnts = _bucket_edges(src, dst)

    agg = _segmax128(x, srcs, dls, cnts)
    h = _linear(agg, x, Wl1, Wr1, b1, "relu")
    for Wl, Wr, b in ((Wl2, Wr2, b2), (Wl3, Wr3, b3), (Wl4, Wr4, b4),
                      (Wl5, Wr5, b5), (Wl6, Wr6, b6)):
        agg = _segmax16(h, srcs, dls, cnts)
        h = _linear(agg, h, Wl, Wr, b, "relu")
    agg = _segmax16(h, srcs, dls, cnts)
    return _linear(agg, h, Wl7, Wr7, b7, "lsm")


# speedup vs baseline: 2.8452x; 2.8452x over previous
import functools

import jax
import jax.numpy as jnp
from jax import lax
from jax.experimental import pallas as pl
from jax.experimental.pallas import tpu as pltpu
from jax.experimental.pallas import tpu_sc as plsc

N = 10000
E = 320000
NW = 32
BS = 320
NPAD = NW * BS
SENT = BS
CH = 2000
CAPW = 325632
KG = 128
KB = 2048

_mesh = plsc.VectorSubcoreMesh(core_axis_name="c", subcore_axis_name="s")


def _wid():
    return lax.axis_index("s") * 2 + lax.axis_index("c")


@functools.partial(
    pl.kernel,
    mesh=_mesh,
    out_type=[
        jax.ShapeDtypeStruct((NW, CAPW), jnp.int32),
        jax.ShapeDtypeStruct((NW, CAPW), jnp.int32),
        jax.ShapeDtypeStruct((NW, 16), jnp.int32),
    ],
    scratch_types=[
        pltpu.VMEM((CH,), jnp.int32),
        pltpu.VMEM((CH,), jnp.int32),
        pltpu.VMEM((4112,), jnp.int32),
        pltpu.VMEM((4112,), jnp.int32),
        pltpu.VMEM((16,), jnp.int32),
    ],
    compiler_params=pltpu.CompilerParams(needs_layout_passes=False),
)
def _bucket_edges(src_hbm, dst_hbm, srcs_out, dls_out, cnts_out,
                  sbuf, dbuf, csrc, cdl, cvec):
    wid = _wid()
    lo = wid * BS
    lane = lax.broadcasted_iota(jnp.int32, (16,), 0)
    zero16 = jnp.zeros((16,), jnp.int32)
    sent16 = jnp.full((16,), SENT, jnp.int32)

    def _init(i, _):
        csrc[pl.ds(i * 16, 16)] = zero16
        cdl[pl.ds(i * 16, 16)] = sent16
        return 0

    lax.fori_loop(0, 4112 // 16, _init, 0)

    def _flush(args):
        cnt, nfl = args
        base = pl.multiple_of((nfl & 1) * 2048, 2048)
        dst = pl.multiple_of(nfl * 2048, 2048)
        pltpu.sync_copy(csrc.at[pl.ds(base, 2048)],
                        srcs_out.at[wid, pl.ds(dst, 2048)])
        pltpu.sync_copy(cdl.at[pl.ds(base, 2048)],
                        dls_out.at[wid, pl.ds(dst, 2048)])
        return cnt, nfl + 1

    def _scan_chunk(c, carry):
        pltpu.sync_copy(src_hbm.at[pl.ds(c * CH, CH)], sbuf)
        pltpu.sync_copy(dst_hbm.at[pl.ds(c * CH, CH)], dbuf)

        def _group(g, carry2):
            cnt, nfl = carry2
            dv = dbuf[pl.ds(g * 16, 16)]
            sv = sbuf[pl.ds(g * 16, 16)]
            dlv = dv - lo
            m = (dlv >= 0) & (dlv < BS)
            incl = jnp.cumsum(m.astype(jnp.int32))
            pos = jnp.where(m, (cnt + incl - 1) & 4095, 4096 + lane)
            plsc.store_scatter(csrc, [pos], sv)
            plsc.store_scatter(cdl, [pos], dlv)
            return cnt + incl[15], nfl

        carry = lax.fori_loop(0, CH // 16, _group, carry)
        cnt, nfl = carry
        cnt, nfl = lax.cond(cnt - nfl * 2048 >= 2048, _flush,
                            lambda a: a, (cnt, nfl))
        cnt, nfl = lax.cond(cnt - nfl * 2048 >= 2048, _flush,
                            lambda a: a, (cnt, nfl))
        return cnt, nfl

    cnt, nfl = lax.fori_loop(0, E // CH, _scan_chunk, (0, 0))

    csrc[pl.ds(cnt & 4095, 16)] = zero16
    cdl[pl.ds(cnt & 4095, 16)] = sent16
    cnt_pad = ((cnt + 15) // 16) * 16
    cnt, nfl = _flush((cnt, nfl))
    cnt, nfl = _flush((cnt, nfl))
    cvec[...] = jnp.where(lane == 0, cnt_pad, 0)
    pltpu.sync_copy(cvec, cnts_out.at[wid])


def _make_segmax(F, KBF, NC):
    @functools.partial(
        pl.kernel,
        mesh=_mesh,
        out_type=jax.ShapeDtypeStruct((NPAD, F), jnp.float32),
        scratch_types=(
            [pltpu.VMEM((KBF,), jnp.int32),
             pltpu.VMEM((KBF,), jnp.int32),
             pltpu.VMEM((KBF, F), jnp.float32)]
            + [pltpu.VMEM((BS + 16, F), jnp.float32) for _ in range(NC)]
            + [pltpu.VMEM((16,), jnp.int32),
               pltpu.SemaphoreType.DMA]
        ),
        compiler_params=pltpu.CompilerParams(needs_layout_passes=False,
                                             use_tc_tiling_on_sc=False),
    )
    def segmax(h_hbm, srcs, dls, cnts, out_hbm, cidx, cdl, rows, *rest):
        acc = rest[:NC]
        cvec, sem = rest[NC], rest[NC + 1]
        wid = _wid()
        lo = wid * BS
        neg16 = jnp.full((16,), -jnp.inf, jnp.float32)

        def _initrow(i, _):
            for c in range(NC):
                for f in range(F // 16):
                    acc[c][i, pl.ds(f * 16, 16)] = neg16
            return 0

        lax.fori_loop(0, BS + 16, _initrow, 0)

        pltpu.sync_copy(cnts.at[wid], cvec)
        total = jnp.max(cvec[...])
        nbig = (total + KBF - 1) // KBF

        def _big(cb, _):
            pltpu.sync_copy(srcs.at[wid, pl.ds(cb * KBF, KBF)], cidx)
            pltpu.sync_copy(dls.at[wid, pl.ds(cb * KBF, KBF)], cdl)
            nsm = jnp.minimum(KBF // KG, (total - cb * KBF + KG - 1) // KG)

            def _fire(k, _):
                pltpu.async_copy(h_hbm.at[cidx.at[pl.ds(k * KG, KG)]],
                                 rows.at[pl.ds(k * KG, KG)], sem)
                return 0

            def _drain(k, _):
                pltpu.make_async_copy(h_hbm.at[pl.ds(0, KG)],
                                      rows.at[pl.ds(k * KG, KG)], sem).wait()
                return 0

            lax.fori_loop(0, nsm, _fire, 0)
            lax.fori_loop(0, nsm, _drain, 0)

            def _edge16(g, _):
                gb = g * 16
                dlv = cdl[pl.ds(gb, 16)]
                for j in range(16):
                    a = acc[j % NC]
                    dl = dlv[j]
                    for f in range(F // 16):
                        cur = a[dl, pl.ds(f * 16, 16)]
                        r = rows[gb + j, pl.ds(f * 16, 16)]
                        a[dl, pl.ds(f * 16, 16)] = jnp.maximum(cur, r)
                return 0

            lax.fori_loop(0, nsm * (KG // 16), _edge16, 0)
            return 0

        lax.fori_loop(0, nbig, _big, 0)

        def _fixrow(i, _):
            for f in range(F // 16):
                v = acc[0][i, pl.ds(f * 16, 16)]
                for c in range(1, NC):
                    v = jnp.maximum(v, acc[c][i, pl.ds(f * 16, 16)])
                acc[0][i, pl.ds(f * 16, 16)] = jnp.where(v == -jnp.inf,
                                                         0.0, v)
            return 0

        lax.fori_loop(0, BS, _fixrow, 0)
        pltpu.sync_copy(acc[0].at[pl.ds(0, BS)], out_hbm.at[pl.ds(lo, BS)])

    return segmax


_segmax128 = _make_segmax(128, 256, 2)
_segmax16 = _make_segmax(16, 2048, 8)


def _linear(agg, h, Wl, Wr, b, act):
    M, F = h.shape
    H = Wl.shape[1]
    BM = 1000

    def body(agg_ref, h_ref, wl_ref, wr_ref, b_ref, o_ref):
        o = jnp.dot(agg_ref[...], wl_ref[...],
                    preferred_element_type=jnp.float32)
        o = o + jnp.dot(h_ref[...], wr_ref[...],
                        preferred_element_type=jnp.float32)
        o = o + b_ref[...]
        if act == "relu":
            o = jnp.maximum(o, 0.0)
        elif act == "lsm":
            mx = jnp.max(o, axis=1, keepdims=True)
            e = jnp.exp(o - mx)
            s = jnp.sum(e, axis=1, keepdims=True)
            o = o - mx - jnp.log(s)
        o_ref[...] = o

    return pl.pallas_call(
        body,
        grid=(M // BM,),
        in_specs=[
            pl.BlockSpec((BM, F), lambda i: (i, 0)),
            pl.BlockSpec((BM, F), lambda i: (i, 0)),
            pl.BlockSpec((F, H), lambda i: (0, 0)),
            pl.BlockSpec((F, H), lambda i: (0, 0)),
            pl.BlockSpec((1, H), lambda i: (0, 0)),
        ],
        out_specs=pl.BlockSpec((BM, H), lambda i: (i, 0)),
        out_shape=jax.ShapeDtypeStruct((M, H), jnp.float32),
    )(agg, h, Wl, Wr, b[None])


def kernel(x, edge_index, Wl1, Wr1, b1, Wl2, Wr2, b2, Wl3, Wr3, b3,
           Wl4, Wr4, b4, Wl5, Wr5, b5, Wl6, Wr6, b6, Wl7, Wr7, b7):
    src = edge_index[0]
    dst = edge_index[1]
    srcs, dls, cnts = _bucket_edges(src, dst)

    agg = _segmax128(x, srcs, dls, cnts)
    h = _linear(agg, x, Wl1, Wr1, b1, "relu")
    for Wl, Wr, b in ((Wl2, Wr2, b2), (Wl3, Wr3, b3), (Wl4, Wr4, b4),
                      (Wl5, Wr5, b5), (Wl6, Wr6, b6)):
        agg = _segmax16(h, srcs, dls, cnts)
        h = _linear(agg, h, Wl, Wr, b, "relu")
    agg = _segmax16(h, srcs, dls, cnts)
    return _linear(agg, h, Wl7, Wr7, b7, "lsm")


# speedup vs baseline: 3.1422x; 1.1044x over previous
import functools

import jax
import jax.numpy as jnp
from jax import lax
from jax.experimental import pallas as pl
from jax.experimental.pallas import tpu as pltpu
from jax.experimental.pallas import tpu_sc as plsc

N = 10000
E = 320000
NW = 32
BS = 320
NPAD = NW * BS
SENT = BS
CH = 2000
CAPW = 325632
KG = 128
KB = 2048

_mesh = plsc.VectorSubcoreMesh(core_axis_name="c", subcore_axis_name="s")


def _wid():
    return lax.axis_index("s") * 2 + lax.axis_index("c")


@functools.partial(
    pl.kernel,
    mesh=_mesh,
    out_type=[
        jax.ShapeDtypeStruct((NW, CAPW), jnp.int32),
        jax.ShapeDtypeStruct((NW, CAPW), jnp.int32),
        jax.ShapeDtypeStruct((NW, 16), jnp.int32),
    ],
    scratch_types=[
        pltpu.VMEM((CH,), jnp.int32),
        pltpu.VMEM((CH,), jnp.int32),
        pltpu.VMEM((4112,), jnp.int32),
        pltpu.VMEM((4112,), jnp.int32),
        pltpu.VMEM((16,), jnp.int32),
    ],
    compiler_params=pltpu.CompilerParams(needs_layout_passes=False),
)
def _bucket_edges(src_hbm, dst_hbm, srcs_out, dls_out, cnts_out,
                  sbuf, dbuf, csrc, cdl, cvec):
    wid = _wid()
    lo = wid * BS
    lane = lax.broadcasted_iota(jnp.int32, (16,), 0)
    zero16 = jnp.zeros((16,), jnp.int32)
    sent16 = jnp.full((16,), SENT, jnp.int32)

    def _init(i, _):
        csrc[pl.ds(i * 16, 16)] = zero16
        cdl[pl.ds(i * 16, 16)] = sent16
        return 0

    lax.fori_loop(0, 4112 // 16, _init, 0)

    def _flush(args):
        cnt, nfl = args
        base = pl.multiple_of((nfl & 1) * 2048, 2048)
        dst = pl.multiple_of(nfl * 2048, 2048)
        pltpu.sync_copy(csrc.at[pl.ds(base, 2048)],
                        srcs_out.at[wid, pl.ds(dst, 2048)])
        pltpu.sync_copy(cdl.at[pl.ds(base, 2048)],
                        dls_out.at[wid, pl.ds(dst, 2048)])
        return cnt, nfl + 1

    def _scan_chunk(c, carry):
        pltpu.sync_copy(src_hbm.at[pl.ds(c * CH, CH)], sbuf)
        pltpu.sync_copy(dst_hbm.at[pl.ds(c * CH, CH)], dbuf)

        def _group5(g, carry2):
            cnt, nfl = carry2
            svs, dlvs, ms, incls = [], [], [], []
            for u in range(5):
                b = g * 80 + u * 16
                dv = dbuf[pl.ds(b, 16)]
                svs.append(sbuf[pl.ds(b, 16)])
                dlv = dv - lo
                dlvs.append(dlv)
                m = (dlv >= 0) & (dlv < BS)
                ms.append(m)
                incls.append(jnp.cumsum(m.astype(jnp.int32)))
            for u in range(5):
                pos = jnp.where(ms[u], (cnt + incls[u] - 1) & 4095,
                                4096 + lane)
                plsc.store_scatter(csrc, [pos], svs[u])
                plsc.store_scatter(cdl, [pos], dlvs[u])
                cnt = cnt + incls[u][15]
            return cnt, nfl

        carry = lax.fori_loop(0, CH // 80, _group5, carry)
        cnt, nfl = carry
        cnt, nfl = lax.cond(cnt - nfl * 2048 >= 2048, _flush,
                            lambda a: a, (cnt, nfl))
        cnt, nfl = lax.cond(cnt - nfl * 2048 >= 2048, _flush,
                            lambda a: a, (cnt, nfl))
        return cnt, nfl

    cnt, nfl = lax.fori_loop(0, E // CH, _scan_chunk, (0, 0))

    csrc[pl.ds(cnt & 4095, 16)] = zero16
    cdl[pl.ds(cnt & 4095, 16)] = sent16
    cnt_pad = ((cnt + 15) // 16) * 16
    cnt, nfl = _flush((cnt, nfl))
    cnt, nfl = _flush((cnt, nfl))
    cvec[...] = jnp.where(lane == 0, cnt_pad, 0)
    pltpu.sync_copy(cvec, cnts_out.at[wid])


def _make_segmax(F, KBF, NC):
    @functools.partial(
        pl.kernel,
        mesh=_mesh,
        out_type=jax.ShapeDtypeStruct((NPAD, F), jnp.float32),
        scratch_types=(
            [pltpu.VMEM((KBF,), jnp.int32),
             pltpu.VMEM((KBF,), jnp.int32),
             pltpu.VMEM((KBF, F), jnp.float32)]
            + [pltpu.VMEM((BS + 16, F), jnp.float32) for _ in range(NC)]
            + [pltpu.VMEM((16,), jnp.int32),
               pltpu.SemaphoreType.DMA]
        ),
        compiler_params=pltpu.CompilerParams(needs_layout_passes=False,
                                             use_tc_tiling_on_sc=False),
    )
    def segmax(h_hbm, srcs, dls, cnts, out_hbm, cidx, cdl, rows, *rest):
        acc = rest[:NC]
        cvec, sem = rest[NC], rest[NC + 1]
        wid = _wid()
        lo = wid * BS
        neg16 = jnp.full((16,), -jnp.inf, jnp.float32)

        def _initrow(i, _):
            for c in range(NC):
                for f in range(F // 16):
                    acc[c][i, pl.ds(f * 16, 16)] = neg16
            return 0

        lax.fori_loop(0, BS + 16, _initrow, 0)

        pltpu.sync_copy(cnts.at[wid], cvec)
        total = jnp.max(cvec[...])
        nbig = (total + KBF - 1) // KBF

        def _big(cb, _):
            pltpu.sync_copy(srcs.at[wid, pl.ds(cb * KBF, KBF)], cidx)
            pltpu.sync_copy(dls.at[wid, pl.ds(cb * KBF, KBF)], cdl)
            nsm = jnp.minimum(KBF // KG, (total - cb * KBF + KG - 1) // KG)

            def _fire(k, _):
                pltpu.async_copy(h_hbm.at[cidx.at[pl.ds(k * KG, KG)]],
                                 rows.at[pl.ds(k * KG, KG)], sem)
                return 0

            def _drain(k, _):
                pltpu.make_async_copy(h_hbm.at[pl.ds(0, KG)],
                                      rows.at[pl.ds(k * KG, KG)], sem).wait()
                return 0

            lax.fori_loop(0, nsm, _fire, 0)
            lax.fori_loop(0, nsm, _drain, 0)

            def _edge16(g, _):
                gb = g * 16
                dlv = cdl[pl.ds(gb, 16)]
                for j in range(16):
                    a = acc[j % NC]
                    dl = dlv[j]
                    for f in range(F // 16):
                        cur = a[dl, pl.ds(f * 16, 16)]
                        r = rows[gb + j, pl.ds(f * 16, 16)]
                        a[dl, pl.ds(f * 16, 16)] = jnp.maximum(cur, r)
                return 0

            lax.fori_loop(0, nsm * (KG // 16), _edge16, 0)
            return 0

        lax.fori_loop(0, nbig, _big, 0)

        def _fixrow(i, _):
            for f in range(F // 16):
                v = acc[0][i, pl.ds(f * 16, 16)]
                for c in range(1, NC):
                    v = jnp.maximum(v, acc[c][i, pl.ds(f * 16, 16)])
                acc[0][i, pl.ds(f * 16, 16)] = jnp.where(v == -jnp.inf,
                                                         0.0, v)
            return 0

        lax.fori_loop(0, BS, _fixrow, 0)
        pltpu.sync_copy(acc[0].at[pl.ds(0, BS)], out_hbm.at[pl.ds(lo, BS)])

    return segmax


_segmax128 = _make_segmax(128, 256, 2)
_segmax16 = _make_segmax(16, 2048, 8)


def _linear(agg, h, Wl, Wr, b, act):
    M, F = h.shape
    H = Wl.shape[1]
    BM = 1000

    def body(agg_ref, h_ref, wl_ref, wr_ref, b_ref, o_ref):
        o = jnp.dot(agg_ref[...], wl_ref[...],
                    preferred_element_type=jnp.float32)
        o = o + jnp.dot(h_ref[...], wr_ref[...],
                        preferred_element_type=jnp.float32)
        o = o + b_ref[...]
        if act == "relu":
            o = jnp.maximum(o, 0.0)
        elif act == "lsm":
            mx = jnp.max(o, axis=1, keepdims=True)
            e = jnp.exp(o - mx)
            s = jnp.sum(e, axis=1, keepdims=True)
            o = o - mx - jnp.log(s)
        o_ref[...] = o

    return pl.pallas_call(
        body,
        grid=(M // BM,),
        in_specs=[
            pl.BlockSpec((BM, F), lambda i: (i, 0)),
            pl.BlockSpec((BM, F), lambda i: (i, 0)),
            pl.BlockSpec((F, H), lambda i: (0, 0)),
            pl.BlockSpec((F, H), lambda i: (0, 0)),
            pl.BlockSpec((1, H), lambda i: (0, 0)),
        ],
        out_specs=pl.BlockSpec((BM, H), lambda i: (i, 0)),
        out_shape=jax.ShapeDtypeStruct((M, H), jnp.float32),
    )(agg, h, Wl, Wr, b[None])


def kernel(x, edge_index, Wl1, Wr1, b1, Wl2, Wr2, b2, Wl3, Wr3, b3,
           Wl4, Wr4, b4, Wl5, Wr5, b5, Wl6, Wr6, b6, Wl7, Wr7, b7):
    src = edge_index[0]
    dst = edge_index[1]
    srcs, dls, cnts = _bucket_edges(src, dst)

    agg = _segmax128(x, srcs, dls, cnts)
    h = _linear(agg, x, Wl1, Wr1, b1, "relu")
    for Wl, Wr, b in ((Wl2, Wr2, b2), (Wl3, Wr3, b3), (Wl4, Wr4, b4),
                      (Wl5, Wr5, b5), (Wl6, Wr6, b6)):
        agg = _segmax16(h, srcs, dls, cnts)
        h = _linear(agg, h, Wl, Wr, b, "relu")
    agg = _segmax16(h, srcs, dls, cnts)
    return _linear(agg, h, Wl7, Wr7, b7, "lsm")


# speedup vs baseline: 3.1979x; 1.0177x over previous
import functools

import jax
import jax.numpy as jnp
from jax import lax
from jax.experimental import pallas as pl
from jax.experimental.pallas import tpu as pltpu
from jax.experimental.pallas import tpu_sc as plsc

N = 10000
E = 320000
NW = 32
BS = 320
NPAD = NW * BS
SENT = BS
CH = 2000
CAPW = 325632
KG = 128
KB = 2048

_mesh = plsc.VectorSubcoreMesh(core_axis_name="c", subcore_axis_name="s")


def _wid():
    return lax.axis_index("s") * 2 + lax.axis_index("c")


@functools.partial(
    pl.kernel,
    mesh=_mesh,
    out_type=[
        jax.ShapeDtypeStruct((NW, CAPW), jnp.int32),
        jax.ShapeDtypeStruct((NW, CAPW), jnp.int32),
        jax.ShapeDtypeStruct((NW, 16), jnp.int32),
    ],
    scratch_types=[
        pltpu.VMEM((CH,), jnp.int32),
        pltpu.VMEM((CH,), jnp.int32),
        pltpu.VMEM((4112,), jnp.int32),
        pltpu.VMEM((4112,), jnp.int32),
        pltpu.VMEM((16,), jnp.int32),
    ],
    compiler_params=pltpu.CompilerParams(needs_layout_passes=False),
)
def _bucket_edges(src_hbm, dst_hbm, srcs_out, dls_out, cnts_out,
                  sbuf, dbuf, csrc, cdl, cvec):
    wid = _wid()
    lo = wid * BS
    lane = lax.broadcasted_iota(jnp.int32, (16,), 0)
    zero16 = jnp.zeros((16,), jnp.int32)
    sent16 = jnp.full((16,), SENT, jnp.int32)

    def _init(i, _):
        csrc[pl.ds(i * 16, 16)] = zero16
        cdl[pl.ds(i * 16, 16)] = sent16
        return 0

    lax.fori_loop(0, 4112 // 16, _init, 0)

    def _flush(args):
        cnt, nfl = args
        base = pl.multiple_of((nfl & 1) * 2048, 2048)
        dst = pl.multiple_of(nfl * 2048, 2048)
        pltpu.sync_copy(csrc.at[pl.ds(base, 2048)],
                        srcs_out.at[wid, pl.ds(dst, 2048)])
        pltpu.sync_copy(cdl.at[pl.ds(base, 2048)],
                        dls_out.at[wid, pl.ds(dst, 2048)])
        return cnt, nfl + 1

    def _scan_chunk(c, carry):
        pltpu.sync_copy(src_hbm.at[pl.ds(c * CH, CH)], sbuf)
        pltpu.sync_copy(dst_hbm.at[pl.ds(c * CH, CH)], dbuf)

        def _group5(g, carry2):
            cnt, nfl = carry2
            svs, dlvs, ms, incls = [], [], [], []
            for u in range(5):
                b = g * 80 + u * 16
                dv = dbuf[pl.ds(b, 16)]
                svs.append(sbuf[pl.ds(b, 16)])
                dlv = dv - lo
                dlvs.append(dlv)
                m = (dlv >= 0) & (dlv < BS)
                ms.append(m)
                incls.append(jnp.cumsum(m.astype(jnp.int32)))
            for u in range(5):
                pos = jnp.where(ms[u], (cnt + incls[u] - 1) & 4095,
                                4096 + lane)
                plsc.store_scatter(csrc, [pos], svs[u])
                plsc.store_scatter(cdl, [pos], dlvs[u])
                cnt = cnt + incls[u][15]
            return cnt, nfl

        carry = lax.fori_loop(0, CH // 80, _group5, carry)
        cnt, nfl = carry
        cnt, nfl = lax.cond(cnt - nfl * 2048 >= 2048, _flush,
                            lambda a: a, (cnt, nfl))
        cnt, nfl = lax.cond(cnt - nfl * 2048 >= 2048, _flush,
                            lambda a: a, (cnt, nfl))
        return cnt, nfl

    cnt, nfl = lax.fori_loop(0, E // CH, _scan_chunk, (0, 0))

    csrc[pl.ds(cnt & 4095, 16)] = zero16
    cdl[pl.ds(cnt & 4095, 16)] = sent16
    cnt_pad = ((cnt + 15) // 16) * 16
    cnt, nfl = _flush((cnt, nfl))
    cnt, nfl = _flush((cnt, nfl))
    cvec[...] = jnp.where(lane == 0, cnt_pad, 0)
    pltpu.sync_copy(cvec, cnts_out.at[wid])


def _make_segmax(F, KBF, NC):
    @functools.partial(
        pl.kernel,
        mesh=_mesh,
        out_type=jax.ShapeDtypeStruct((NPAD, F), jnp.float32),
        scratch_types=(
            [pltpu.VMEM((KBF,), jnp.int32),
             pltpu.VMEM((KBF,), jnp.int32),
             pltpu.VMEM((KBF, F), jnp.float32)]
            + [pltpu.VMEM((BS + 16, F), jnp.float32) for _ in range(NC)]
            + [pltpu.VMEM((16,), jnp.int32),
               pltpu.SemaphoreType.DMA]
        ),
        compiler_params=pltpu.CompilerParams(needs_layout_passes=False,
                                             use_tc_tiling_on_sc=False),
    )
    def segmax(h_hbm, srcs, dls, cnts, out_hbm, cidx, cdl, rows, *rest):
        acc = rest[:NC]
        cvec, sem = rest[NC], rest[NC + 1]
        wid = _wid()
        lo = wid * BS
        neg16 = jnp.full((16,), -jnp.inf, jnp.float32)

        def _initrow(i, _):
            for c in range(NC):
                for f in range(F // 16):
                    acc[c][i, pl.ds(f * 16, 16)] = neg16
            return 0

        lax.fori_loop(0, BS + 16, _initrow, 0)

        pltpu.sync_copy(cnts.at[wid], cvec)
        total = jnp.max(cvec[...])
        nbig = (total + KBF - 1) // KBF

        def _big(cb, _):
            pltpu.sync_copy(srcs.at[wid, pl.ds(cb * KBF, KBF)], cidx)
            pltpu.sync_copy(dls.at[wid, pl.ds(cb * KBF, KBF)], cdl)
            nsm = jnp.minimum(KBF // KG, (total - cb * KBF + KG - 1) // KG)

            def _fire(k, _):
                pltpu.async_copy(h_hbm.at[cidx.at[pl.ds(k * KG, KG)]],
                                 rows.at[pl.ds(k * KG, KG)], sem)
                return 0

            def _drain(k, _):
                pltpu.make_async_copy(h_hbm.at[pl.ds(0, KG)],
                                      rows.at[pl.ds(k * KG, KG)], sem).wait()
                return 0

            lax.fori_loop(0, nsm, _fire, 0)
            lax.fori_loop(0, nsm, _drain, 0)

            def _edge32(g, _):
                gb = g * 32
                dlv0 = cdl[pl.ds(gb, 16)]
                dlv1 = cdl[pl.ds(gb + 16, 16)]
                for j in range(32):
                    a = acc[j % NC]
                    dl = dlv0[j] if j < 16 else dlv1[j - 16]
                    for f in range(F // 16):
                        cur = a[dl, pl.ds(f * 16, 16)]
                        r = rows[gb + j, pl.ds(f * 16, 16)]
                        a[dl, pl.ds(f * 16, 16)] = jnp.maximum(cur, r)
                return 0

            lax.fori_loop(0, nsm * (KG // 32), _edge32, 0)
            return 0

        lax.fori_loop(0, nbig, _big, 0)

        def _fixrow(i, _):
            for f in range(F // 16):
                v = acc[0][i, pl.ds(f * 16, 16)]
                for c in range(1, NC):
                    v = jnp.maximum(v, acc[c][i, pl.ds(f * 16, 16)])
                acc[0][i, pl.ds(f * 16, 16)] = jnp.where(v == -jnp.inf,
                                                         0.0, v)
            return 0

        lax.fori_loop(0, BS, _fixrow, 0)
        pltpu.sync_copy(acc[0].at[pl.ds(0, BS)], out_hbm.at[pl.ds(lo, BS)])

    return segmax


_segmax128 = _make_segmax(128, 256, 2)
_segmax16 = _make_segmax(16, 2048, 8)


def _linear(agg, h, Wl, Wr, b, act):
    M, F = h.shape
    H = Wl.shape[1]
    BM = 1000

    def body(agg_ref, h_ref, wl_ref, wr_ref, b_ref, o_ref):
        o = jnp.dot(agg_ref[...], wl_ref[...],
                    preferred_element_type=jnp.float32)
        o = o + jnp.dot(h_ref[...], wr_ref[...],
                        preferred_element_type=jnp.float32)
        o = o + b_ref[...]
        if act == "relu":
            o = jnp.maximum(o, 0.0)
        elif act == "lsm":
            mx = jnp.max(o, axis=1, keepdims=True)
            e = jnp.exp(o - mx)
            s = jnp.sum(e, axis=1, keepdims=True)
            o = o - mx - jnp.log(s)
        o_ref[...] = o

    return pl.pallas_call(
        body,
        grid=(M // BM,),
        in_specs=[
            pl.BlockSpec((BM, F), lambda i: (i, 0)),
            pl.BlockSpec((BM, F), lambda i: (i, 0)),
            pl.BlockSpec((F, H), lambda i: (0, 0)),
            pl.BlockSpec((F, H), lambda i: (0, 0)),
            pl.BlockSpec((1, H), lambda i: (0, 0)),
        ],
        out_specs=pl.BlockSpec((BM, H), lambda i: (i, 0)),
        out_shape=jax.ShapeDtypeStruct((M, H), jnp.float32),
    )(agg, h, Wl, Wr, b[None])


def kernel(x, edge_index, Wl1, Wr1, b1, Wl2, Wr2, b2, Wl3, Wr3, b3,
           Wl4, Wr4, b4, Wl5, Wr5, b5, Wl6, Wr6, b6, Wl7, Wr7, b7):
    src = edge_index[0]
    dst = edge_index[1]
    srcs, dls, cnts = _bucket_edges(src, dst)

    agg = _segmax128(x, srcs, dls, cnts)
    h = _linear(agg, x, Wl1, Wr1, b1, "relu")
    for Wl, Wr, b in ((Wl2, Wr2, b2), (Wl3, Wr3, b3), (Wl4, Wr4, b4),
                      (Wl5, Wr5, b5), (Wl6, Wr6, b6)):
        agg = _segmax16(h, srcs, dls, cnts)
        h = _linear(agg, h, Wl, Wr, b, "relu")
    agg = _segmax16(h, srcs, dls, cnts)
    return _linear(agg, h, Wl7, Wr7, b7, "lsm")


# speedup vs baseline: 3.1999x; 1.0006x over previous
import functools

import jax
import jax.numpy as jnp
from jax import lax
from jax.experimental import pallas as pl
from jax.experimental.pallas import tpu as pltpu
from jax.experimental.pallas import tpu_sc as plsc

N = 10000
E = 320000
NW = 32
BS = 320
NPAD = NW * BS
SENT = BS
CH = 2000
CAPW = 325632
KG = 128
KB = 2048

_mesh = plsc.VectorSubcoreMesh(core_axis_name="c", subcore_axis_name="s")


def _wid():
    return lax.axis_index("s") * 2 + lax.axis_index("c")


@functools.partial(
    pl.kernel,
    mesh=_mesh,
    out_type=[
        jax.ShapeDtypeStruct((NW, CAPW), jnp.int32),
        jax.ShapeDtypeStruct((NW, CAPW), jnp.int32),
        jax.ShapeDtypeStruct((NW, 16), jnp.int32),
    ],
    scratch_types=[
        pltpu.VMEM((CH,), jnp.int32),
        pltpu.VMEM((CH,), jnp.int32),
        pltpu.VMEM((4112,), jnp.int32),
        pltpu.VMEM((4112,), jnp.int32),
        pltpu.VMEM((16,), jnp.int32),
    ],
    compiler_params=pltpu.CompilerParams(needs_layout_passes=False),
)
def _bucket_edges(src_hbm, dst_hbm, srcs_out, dls_out, cnts_out,
                  sbuf, dbuf, csrc, cdl, cvec):
    wid = _wid()
    lo = wid * BS
    lane = lax.broadcasted_iota(jnp.int32, (16,), 0)
    zero16 = jnp.zeros((16,), jnp.int32)
    sent16 = jnp.full((16,), SENT, jnp.int32)

    def _init(i, _):
        csrc[pl.ds(i * 16, 16)] = zero16
        cdl[pl.ds(i * 16, 16)] = sent16
        return 0

    lax.fori_loop(0, 4112 // 16, _init, 0)

    def _flush(args):
        cnt, nfl = args
        base = pl.multiple_of((nfl & 1) * 2048, 2048)
        dst = pl.multiple_of(nfl * 2048, 2048)
        pltpu.sync_copy(csrc.at[pl.ds(base, 2048)],
                        srcs_out.at[wid, pl.ds(dst, 2048)])
        pltpu.sync_copy(cdl.at[pl.ds(base, 2048)],
                        dls_out.at[wid, pl.ds(dst, 2048)])
        return cnt, nfl + 1

    def _scan_chunk(c, carry):
        pltpu.sync_copy(src_hbm.at[pl.ds(c * CH, CH)], sbuf)
        pltpu.sync_copy(dst_hbm.at[pl.ds(c * CH, CH)], dbuf)

        def _group5(g, carry2):
            cnt, nfl = carry2
            svs, dlvs, ms, incls = [], [], [], []
            for u in range(5):
                b = g * 80 + u * 16
                dv = dbuf[pl.ds(b, 16)]
                svs.append(sbuf[pl.ds(b, 16)])
                dlv = dv - lo
                dlvs.append(dlv)
                m = (dlv >= 0) & (dlv < BS)
                ms.append(m)
                incls.append(jnp.cumsum(m.astype(jnp.int32)))
            for u in range(5):
                pos = jnp.where(ms[u], (cnt + incls[u] - 1) & 4095,
                                4096 + lane)
                plsc.store_scatter(csrc, [pos], svs[u])
                plsc.store_scatter(cdl, [pos], dlvs[u])
                cnt = cnt + incls[u][15]
            return cnt, nfl

        carry = lax.fori_loop(0, CH // 80, _group5, carry)
        cnt, nfl = carry
        cnt, nfl = lax.cond(cnt - nfl * 2048 >= 2048, _flush,
                            lambda a: a, (cnt, nfl))
        cnt, nfl = lax.cond(cnt - nfl * 2048 >= 2048, _flush,
                            lambda a: a, (cnt, nfl))
        return cnt, nfl

    cnt, nfl = lax.fori_loop(0, E // CH, _scan_chunk, (0, 0))

    csrc[pl.ds(cnt & 4095, 16)] = zero16
    cdl[pl.ds(cnt & 4095, 16)] = sent16
    cnt_pad = ((cnt + 15) // 16) * 16
    cnt, nfl = _flush((cnt, nfl))
    cnt, nfl = _flush((cnt, nfl))
    cvec[...] = jnp.where(lane == 0, cnt_pad, 0)
    pltpu.sync_copy(cvec, cnts_out.at[wid])


def _make_segmax(F, KBF, NC):
    @functools.partial(
        pl.kernel,
        mesh=_mesh,
        out_type=jax.ShapeDtypeStruct((NPAD, F), jnp.float32),
        scratch_types=(
            [pltpu.VMEM((KBF,), jnp.int32),
             pltpu.VMEM((KBF,), jnp.int32),
             pltpu.VMEM((KBF, F), jnp.float32)]
            + [pltpu.VMEM((BS + 16, F), jnp.float32) for _ in range(NC)]
            + [pltpu.VMEM((16,), jnp.int32),
               pltpu.SemaphoreType.DMA]
        ),
        compiler_params=pltpu.CompilerParams(needs_layout_passes=False,
                                             use_tc_tiling_on_sc=False),
    )
    def segmax(h_hbm, srcs, dls, cnts, out_hbm, cidx, cdl, rows, *rest):
        acc = rest[:NC]
        cvec, sem = rest[NC], rest[NC + 1]
        wid = _wid()
        lo = wid * BS
        neg16 = jnp.full((16,), -jnp.inf, jnp.float32)

        def _initrow(i, _):
            for c in range(NC):
                for f in range(F // 16):
                    acc[c][i, pl.ds(f * 16, 16)] = neg16
            return 0

        lax.fori_loop(0, BS + 16, _initrow, 0)

        pltpu.sync_copy(cnts.at[wid], cvec)
        total = jnp.max(cvec[...])
        nbig = (total + KBF - 1) // KBF

        def _big(cb, _):
            pltpu.sync_copy(srcs.at[wid, pl.ds(cb * KBF, KBF)], cidx)
            pltpu.sync_copy(dls.at[wid, pl.ds(cb * KBF, KBF)], cdl)
            nsm = jnp.minimum(KBF // KG, (total - cb * KBF + KG - 1) // KG)

            def _fire(k, _):
                pltpu.async_copy(h_hbm.at[cidx.at[pl.ds(k * KG, KG)]],
                                 rows.at[pl.ds(k * KG, KG)], sem)
                return 0

            def _drain(k, _):
                pltpu.make_async_copy(h_hbm.at[cidx.at[pl.ds(k * KG, KG)]],
                                      rows.at[pl.ds(k * KG, KG)], sem).wait()
                return 0

            lax.fori_loop(0, nsm, _fire, 0)
            lax.fori_loop(0, nsm, _drain, 0)

            def _edge32(g, _):
                gb = g * 32
                dlv0 = cdl[pl.ds(gb, 16)]
                dlv1 = cdl[pl.ds(gb + 16, 16)]
                for j in range(32):
                    a = acc[j % NC]
                    dl = dlv0[j] if j < 16 else dlv1[j - 16]
                    for f in range(F // 16):
                        cur = a[dl, pl.ds(f * 16, 16)]
                        r = rows[gb + j, pl.ds(f * 16, 16)]
                        a[dl, pl.ds(f * 16, 16)] = jnp.maximum(cur, r)
                return 0

            lax.fori_loop(0, nsm * (KG // 32), _edge32, 0)
            return 0

        lax.fori_loop(0, nbig, _big, 0)

        def _fixrow(i, _):
            for f in range(F // 16):
                v = acc[0][i, pl.ds(f * 16, 16)]
                for c in range(1, NC):
                    v = jnp.maximum(v, acc[c][i, pl.ds(f * 16, 16)])
                acc[0][i, pl.ds(f * 16, 16)] = jnp.where(v == -jnp.inf,
                                                         0.0, v)
            return 0

        lax.fori_loop(0, BS, _fixrow, 0)
        pltpu.sync_copy(acc[0].at[pl.ds(0, BS)], out_hbm.at[pl.ds(lo, BS)])

    return segmax


_segmax128 = _make_segmax(128, 256, 2)
_segmax16 = _make_segmax(16, 2048, 8)


def _linear(agg, h, Wl, Wr, b, act):
    M, F = h.shape
    H = Wl.shape[1]
    BM = 1000

    def body(agg_ref, h_ref, wl_ref, wr_ref, b_ref, o_ref):
        o = jnp.dot(agg_ref[...], wl_ref[...],
                    preferred_element_type=jnp.float32)
        o = o + jnp.dot(h_ref[...], wr_ref[...],
                        preferred_element_type=jnp.float32)
        o = o + b_ref[...]
        if act == "relu":
            o = jnp.maximum(o, 0.0)
        elif act == "lsm":
            mx = jnp.max(o, axis=1, keepdims=True)
            e = jnp.exp(o - mx)
            s = jnp.sum(e, axis=1, keepdims=True)
            o = o - mx - jnp.log(s)
        o_ref[...] = o

    return pl.pallas_call(
        body,
        grid=(M // BM,),
        in_specs=[
            pl.BlockSpec((BM, F), lambda i: (i, 0)),
            pl.BlockSpec((BM, F), lambda i: (i, 0)),
            pl.BlockSpec((F, H), lambda i: (0, 0)),
            pl.BlockSpec((F, H), lambda i: (0, 0)),
            pl.BlockSpec((1, H), lambda i: (0, 0)),
        ],
        out_specs=pl.BlockSpec((BM, H), lambda i: (i, 0)),
        out_shape=jax.ShapeDtypeStruct((M, H), jnp.float32),
    )(agg, h, Wl, Wr, b[None])


def kernel(x, edge_index, Wl1, Wr1, b1, Wl2, Wr2, b2, Wl3, Wr3, b3,
           Wl4, Wr4, b4, Wl5, Wr5, b5, Wl6, Wr6, b6, Wl7, Wr7, b7):
    src = edge_index[0]
    dst = edge_index[1]
    srcs, dls, cnts = _bucket_edges(src, dst)

    agg = _segmax128(x, srcs, dls, cnts)
    h = _linear(agg, x, Wl1, Wr1, b1, "relu")
    for Wl, Wr, b in ((Wl2, Wr2, b2), (Wl3, Wr3, b3), (Wl4, Wr4, b4),
                      (Wl5, Wr5, b5), (Wl6, Wr6, b6)):
        agg = _segmax16(h, srcs, dls, cnts)
        h = _linear(agg, h, Wl, Wr, b, "relu")
    agg = _segmax16(h, srcs, dls, cnts)
    return _linear(agg, h, Wl7, Wr7, b7, "lsm")
